# Initial kernel scaffold; baseline (speedup 1.0000x reference)
import jax
import jax.numpy as jnp
from jax.experimental import pallas as pl


def kernel(inp, edge_index, weight1, gcn1_weight, gcn2_weight, weight2):
    raise NotImplementedError("write your pallas kernel here")


# speedup vs baseline: 6.5881x; 6.5881x over previous
import functools

import jax
import jax.numpy as jnp
from jax import lax
from jax.experimental import pallas as pl
from jax.experimental.pallas import tpu as pltpu
from jax.experimental.pallas import tpu_sc as plsc

_N = 10000
_E = 320000
_H1 = 64
_H2 = 32
_H3 = 16
_OUT = 3

_NC = 2
_NS = 16
_NW = _NC * _NS
_EPW = _E // _NW
_CH = 80
_NCHUNK = _EPW // _CH
_RPT = 632
_NP = _RPT * _NS

_ROWS_A = 256
_ROWS_B = _RPT


def _dense_in_body(inp_ref, w1_ref, g1_ref, x_ref, s1_ref):
    x = jnp.dot(inp_ref[...], w1_ref[...], preferred_element_type=jnp.float32)
    x_ref[...] = x
    s1_ref[...] = jnp.dot(x, g1_ref[...], preferred_element_type=jnp.float32)


def _mid_body(p1_ref, g2_ref, gnn1_ref, s2_ref):
    g = jnp.tanh(p1_ref[0] + p1_ref[1])
    gnn1_ref[...] = g
    s2_ref[...] = jnp.dot(g, g2_ref[...], preferred_element_type=jnp.float32)


def _out_body(x_ref, gnn1_ref, p2_ref, w2_ref, out_ref):
    gnn2 = p2_ref[0] + p2_ref[1]
    acc = jnp.dot(x_ref[...], w2_ref[:_H1, :], preferred_element_type=jnp.float32)
    acc = acc + jnp.dot(gnn1_ref[...], w2_ref[_H1:_H1 + _H2, :],
                        preferred_element_type=jnp.float32)
    acc = acc + jnp.dot(gnn2, w2_ref[_H1 + _H2:, :],
                        preferred_element_type=jnp.float32)
    out_ref[...] = acc


def _spmm_partials(src3, dst3, table, zeros, feat):
    mesh = plsc.VectorSubcoreMesh(core_axis_name="c", subcore_axis_name="s")

    @functools.partial(
        pl.kernel,
        mesh=mesh,
        out_type=jax.ShapeDtypeStruct((_NC, _NP, feat), jnp.float32),
        scratch_types=[
            pltpu.VMEM((_NCHUNK, _CH), jnp.int32),
            pltpu.VMEM((_NCHUNK, _CH), jnp.int32),
            pltpu.VMEM((_CH, feat), jnp.float32),
            pltpu.VMEM((_RPT, feat), jnp.float32),
            pltpu.VMEM_SHARED((_NP, feat), jnp.float32),
            pltpu.SemaphoreType.DMA,
        ],
        compiler_params=pltpu.CompilerParams(use_tc_tiling_on_sc=False),
    )
    def spmm(src_hbm, dst_hbm, table_hbm, zeros_hbm, out_hbm,
             src_v, dst_v, rows_v, stage_v, acc_sh, sem):
        cid = lax.axis_index("c")
        sid = lax.axis_index("s")
        wid = sid * _NC + cid
        rows = pl.ds(sid * _RPT, _RPT)
        pltpu.sync_copy(zeros_hbm.at[rows], stage_v)
        pltpu.sync_copy(stage_v, acc_sh.at[rows])
        pltpu.sync_copy(src_hbm.at[wid], src_v)
        pltpu.sync_copy(dst_hbm.at[wid], dst_v)
        plsc.subcore_barrier()

        def body(j, carry):
            pltpu.async_copy(table_hbm.at[src_v.at[j]], rows_v, sem).wait()
            pltpu.sync_copy(rows_v, acc_sh.at[dst_v.at[j]], add=True)
            return carry

        lax.fori_loop(0, _NCHUNK, body, 0)
        plsc.subcore_barrier()
        pltpu.sync_copy(acc_sh.at[rows], stage_v)
        pltpu.sync_copy(stage_v, out_hbm.at[cid, rows])

    return spmm(src3, dst3, table, zeros)


def _dense_in(inp, weight1, gcn1_weight):
    grid = (pl.cdiv(_N, _ROWS_A),)
    return pl.pallas_call(
        _dense_in_body,
        grid=grid,
        in_specs=[
            pl.BlockSpec((_ROWS_A, _N), lambda i: (i, 0)),
            pl.BlockSpec((_N, _H1), lambda i: (0, 0)),
            pl.BlockSpec((_H1, _H2), lambda i: (0, 0)),
        ],
        out_specs=[
            pl.BlockSpec((_ROWS_A, _H1), lambda i: (i, 0)),
            pl.BlockSpec((_ROWS_A, _H2), lambda i: (i, 0)),
        ],
        out_shape=[
            jax.ShapeDtypeStruct((_N, _H1), jnp.float32),
            jax.ShapeDtypeStruct((_NP, _H2), jnp.float32),
        ],
    )(inp, weight1, gcn1_weight)


def _mid(p1, gcn2_weight):
    grid = (_NP // _ROWS_B,)
    return pl.pallas_call(
        _mid_body,
        grid=grid,
        in_specs=[
            pl.BlockSpec((_NC, _ROWS_B, _H2), lambda i: (0, i, 0)),
            pl.BlockSpec((_H2, _H3), lambda i: (0, 0)),
        ],
        out_specs=[
            pl.BlockSpec((_ROWS_B, _H2), lambda i: (i, 0)),
            pl.BlockSpec((_ROWS_B, _H3), lambda i: (i, 0)),
        ],
        out_shape=[
            jax.ShapeDtypeStruct((_N, _H2), jnp.float32),
            jax.ShapeDtypeStruct((_NP, _H3), jnp.float32),
        ],
    )(p1, gcn2_weight)


def _final(x, gnn1, p2, weight2):
    grid = (_NP // _ROWS_B,)
    return pl.pallas_call(
        _out_body,
        grid=grid,
        in_specs=[
            pl.BlockSpec((_ROWS_B, _H1), lambda i: (i, 0)),
            pl.BlockSpec((_ROWS_B, _H2), lambda i: (i, 0)),
            pl.BlockSpec((_NC, _ROWS_B, _H3), lambda i: (0, i, 0)),
            pl.BlockSpec((_H1 + _H2 + _H3, _OUT), lambda i: (0, 0)),
        ],
        out_specs=pl.BlockSpec((_ROWS_B, _OUT), lambda i: (i, 0)),
        out_shape=jax.ShapeDtypeStruct((_N, _OUT), jnp.float32),
    )(x, gnn1, p2, weight2)


def kernel(inp, edge_index, weight1, gcn1_weight, gcn2_weight, weight2):
    src3 = edge_index[0].astype(jnp.int32).reshape(_NW, _NCHUNK, _CH)
    dst3 = edge_index[1].astype(jnp.int32).reshape(_NW, _NCHUNK, _CH)
    zeros2 = jnp.zeros((_NP, _H2), jnp.float32)
    zeros3 = jnp.zeros((_NP, _H3), jnp.float32)

    x, s1 = _dense_in(inp, weight1, gcn1_weight)
    p1 = _spmm_partials(src3, dst3, s1, zeros2, _H2)
    gnn1, s2 = _mid(p1, gcn2_weight)
    p2 = _spmm_partials(src3, dst3, s2, zeros3, _H3)
    return _final(x, gnn1, p2, weight2)


# speedup vs baseline: 10.1203x; 1.5362x over previous
import functools

import jax
import jax.numpy as jnp
from jax import lax
from jax.experimental import pallas as pl
from jax.experimental.pallas import tpu as pltpu
from jax.experimental.pallas import tpu_sc as plsc

_N = 10000
_E = 320000
_H1 = 64
_H2 = 32
_H3 = 16
_OUT = 3

_NC = 2
_NS = 16
_NW = _NC * _NS
_EPW = _E // _NW
_CH = 80
_NCHUNK = _EPW // _CH
_NBUF = 5
_RPT = 632
_NP = _RPT * _NS

_ROWS_A = 256
_ROWS_B = _RPT


def _dense_in_body(inp_ref, w1_ref, g1_ref, x_ref, s1_ref):
    x = jnp.dot(inp_ref[...], w1_ref[...], preferred_element_type=jnp.float32)
    x_ref[...] = x
    s1_ref[...] = jnp.dot(x, g1_ref[...], preferred_element_type=jnp.float32)


def _mid_body(p1_ref, g2_ref, gnn1_ref, s2_ref):
    g = jnp.tanh(p1_ref[0] + p1_ref[1])
    gnn1_ref[...] = g
    s2_ref[...] = jnp.dot(g, g2_ref[...], preferred_element_type=jnp.float32)


def _out_body(x_ref, gnn1_ref, p2_ref, w2_ref, out_ref):
    gnn2 = p2_ref[0] + p2_ref[1]
    acc = jnp.dot(x_ref[...], w2_ref[:_H1, :], preferred_element_type=jnp.float32)
    acc = acc + jnp.dot(gnn1_ref[...], w2_ref[_H1:_H1 + _H2, :],
                        preferred_element_type=jnp.float32)
    acc = acc + jnp.dot(gnn2, w2_ref[_H1 + _H2:, :],
                        preferred_element_type=jnp.float32)
    out_ref[...] = acc


def _spmm_partials(src3, dst3, table, zeros, feat):
    mesh = plsc.VectorSubcoreMesh(core_axis_name="c", subcore_axis_name="s")

    @functools.partial(
        pl.kernel,
        mesh=mesh,
        out_type=jax.ShapeDtypeStruct((_NC, _NP, feat), jnp.float32),
        scratch_types=[
            pltpu.VMEM((_NCHUNK, _CH), jnp.int32),
            pltpu.VMEM((_NCHUNK, _CH), jnp.int32),
            [pltpu.VMEM((_CH, feat), jnp.float32) for _ in range(_NBUF)],
            pltpu.VMEM((_RPT, feat), jnp.float32),
            pltpu.VMEM_SHARED((_NP, feat), jnp.float32),
            [pltpu.SemaphoreType.DMA for _ in range(_NBUF)],
        ],
        compiler_params=pltpu.CompilerParams(use_tc_tiling_on_sc=False),
    )
    def spmm(src_hbm, dst_hbm, table_hbm, zeros_hbm, out_hbm,
             src_v, dst_v, rows_v, stage_v, acc_sh, sems):
        cid = lax.axis_index("c")
        sid = lax.axis_index("s")
        wid = sid * _NC + cid
        rows = pl.ds(sid * _RPT, _RPT)
        pltpu.sync_copy(zeros_hbm.at[rows], stage_v)
        pltpu.sync_copy(stage_v, acc_sh.at[rows])
        pltpu.sync_copy(src_hbm.at[wid], src_v)
        pltpu.sync_copy(dst_hbm.at[wid], dst_v)
        plsc.subcore_barrier()

        for b in range(_NBUF):
            pltpu.async_copy(table_hbm.at[src_v.at[b]], rows_v[b], sems[b])

        def body(g, carry):
            for b in range(_NBUF):
                j = g * _NBUF + b
                pltpu.make_async_copy(table_hbm.at[src_v.at[j]],
                                      rows_v[b], sems[b]).wait()
                pltpu.sync_copy(rows_v[b], acc_sh.at[dst_v.at[j]], add=True)

                @pl.when(j + _NBUF < _NCHUNK)
                def _():
                    pltpu.async_copy(table_hbm.at[src_v.at[j + _NBUF]],
                                     rows_v[b], sems[b])
            return carry

        lax.fori_loop(0, _NCHUNK // _NBUF, body, 0)
        plsc.subcore_barrier()
        pltpu.sync_copy(acc_sh.at[rows], stage_v)
        pltpu.sync_copy(stage_v, out_hbm.at[cid, rows])

    return spmm(src3, dst3, table, zeros)


def _dense_in(inp, weight1, gcn1_weight):
    grid = (pl.cdiv(_N, _ROWS_A),)
    return pl.pallas_call(
        _dense_in_body,
        grid=grid,
        in_specs=[
            pl.BlockSpec((_ROWS_A, _N), lambda i: (i, 0)),
            pl.BlockSpec((_N, _H1), lambda i: (0, 0)),
            pl.BlockSpec((_H1, _H2), lambda i: (0, 0)),
        ],
        out_specs=[
            pl.BlockSpec((_ROWS_A, _H1), lambda i: (i, 0)),
            pl.BlockSpec((_ROWS_A, _H2), lambda i: (i, 0)),
        ],
        out_shape=[
            jax.ShapeDtypeStruct((_N, _H1), jnp.float32),
            jax.ShapeDtypeStruct((_NP, _H2), jnp.float32),
        ],
    )(inp, weight1, gcn1_weight)


def _mid(p1, gcn2_weight):
    grid = (_NP // _ROWS_B,)
    return pl.pallas_call(
        _mid_body,
        grid=grid,
        in_specs=[
            pl.BlockSpec((_NC, _ROWS_B, _H2), lambda i: (0, i, 0)),
            pl.BlockSpec((_H2, _H3), lambda i: (0, 0)),
        ],
        out_specs=[
            pl.BlockSpec((_ROWS_B, _H2), lambda i: (i, 0)),
            pl.BlockSpec((_ROWS_B, _H3), lambda i: (i, 0)),
        ],
        out_shape=[
            jax.ShapeDtypeStruct((_N, _H2), jnp.float32),
            jax.ShapeDtypeStruct((_NP, _H3), jnp.float32),
        ],
    )(p1, gcn2_weight)


def _final(x, gnn1, p2, weight2):
    grid = (_NP // _ROWS_B,)
    return pl.pallas_call(
        _out_body,
        grid=grid,
        in_specs=[
            pl.BlockSpec((_ROWS_B, _H1), lambda i: (i, 0)),
            pl.BlockSpec((_ROWS_B, _H2), lambda i: (i, 0)),
            pl.BlockSpec((_NC, _ROWS_B, _H3), lambda i: (0, i, 0)),
            pl.BlockSpec((_H1 + _H2 + _H3, _OUT), lambda i: (0, 0)),
        ],
        out_specs=pl.BlockSpec((_ROWS_B, _OUT), lambda i: (i, 0)),
        out_shape=jax.ShapeDtypeStruct((_N, _OUT), jnp.float32),
    )(x, gnn1, p2, weight2)


def kernel(inp, edge_index, weight1, gcn1_weight, gcn2_weight, weight2):
    src3 = edge_index[0].astype(jnp.int32).reshape(_NW, _NCHUNK, _CH)
    dst3 = edge_index[1].astype(jnp.int32).reshape(_NW, _NCHUNK, _CH)
    zeros2 = jnp.zeros((_NP, _H2), jnp.float32)
    zeros3 = jnp.zeros((_NP, _H3), jnp.float32)

    x, s1 = _dense_in(inp, weight1, gcn1_weight)
    p1 = _spmm_partials(src3, dst3, s1, zeros2, _H2)
    gnn1, s2 = _mid(p1, gcn2_weight)
    p2 = _spmm_partials(src3, dst3, s2, zeros3, _H3)
    return _final(x, gnn1, p2, weight2)
